# baseline (speedup 1.0000x reference)
import functools

import jax
import jax.numpy as jnp
from jax import lax
from jax.experimental import pallas as pl
from jax.experimental.pallas import tpu as pltpu
from jax.experimental.pallas import tpu_sc as plsc

M, N, DIM = 128, 128, 256
ALPHA = 0.3
SIGMA = max(M, N) / 2.0
NUM_EPOCHS = 100

ROWS = M * N
NC, NS, NLANE = 2, 16, 16
NW = NC * NS
RPW = ROWS // NW
NCH = DIM // NLANE
CH1 = 128
BI = 16
WCOL = N * DIM


def _mesh():
    return plsc.VectorSubcoreMesh(core_axis_name="c", subcore_axis_name="s")


@functools.partial(
    pl.kernel,
    mesh=_mesh(),
    compiler_params=pltpu.CompilerParams(needs_layout_passes=False),
    out_type=[
        jax.ShapeDtypeStruct((NW, NLANE), jnp.float32),
        jax.ShapeDtypeStruct((NW, NLANE), jnp.int32),
    ],
    scratch_types=[
        pltpu.VMEM((DIM,), jnp.float32),
        pltpu.VMEM((CH1, DIM), jnp.float32),
        pltpu.VMEM((CH1, DIM), jnp.float32),
        pltpu.VMEM((NLANE,), jnp.float32),
        pltpu.VMEM((NLANE,), jnp.int32),
        pltpu.SemaphoreType.DMA,
        pltpu.SemaphoreType.DMA,
    ],
)
def _sc_search(w_hbm, x_hbm, dist_out, idx_out, xv, buf0, buf1, sd, si,
               sem0, sem1):
    wid = lax.axis_index("s") * NC + lax.axis_index("c")
    base = wid * RPW
    pltpu.sync_copy(x_hbm, xv)
    xs = [xv[pl.ds(c * NLANE, NLANE)] for c in range(NCH)]
    bufs = (buf0, buf1)
    sems = (sem0, sem1)
    nchunk = RPW // CH1
    cps = [None] * nchunk
    cps[0] = pltpu.async_copy(w_hbm.at[pl.ds(base, CH1)], buf0, sem0)
    best = (jnp.float32(jnp.inf), jnp.int32(0))
    for k in range(nchunk):
        if k + 1 < nchunk:
            cps[k + 1] = pltpu.async_copy(
                w_hbm.at[pl.ds(base + (k + 1) * CH1, CH1)],
                bufs[(k + 1) % 2], sems[(k + 1) % 2])
        cps[k].wait()
        buf = bufs[k % 2]

        def row_body(r, carry, _k=k, _buf=buf):
            bd, bi = carry
            accs = [jnp.zeros((NLANE,), jnp.float32) for _ in range(4)]
            for c in range(NCH):
                d = _buf[r, pl.ds(c * NLANE, NLANE)] - xs[c]
                accs[c % 4] = accs[c % 4] + d * d
            acc = (accs[0] + accs[1]) + (accs[2] + accs[3])
            s = plsc.cumsum(acc)[NLANE - 1]
            better = s < bd
            gi = base + _k * CH1 + r
            return (jnp.where(better, s, bd), jnp.where(better, gi, bi))

        best = lax.fori_loop(0, CH1, row_body, best, unroll=4)
    sd[...] = jnp.full((NLANE,), best[0], jnp.float32)
    si[...] = jnp.full((NLANE,), best[1], jnp.int32)
    pltpu.sync_copy(sd, dist_out.at[wid])
    pltpu.sync_copy(si, idx_out.at[wid])


def _tc_update_body(cd, ci, xr, pr, wr, out, bmu_out, bsm, rj_ref):
    g = pl.program_id(0)
    alpha_op = pr[0, 0]
    inv_sig2 = pr[0, 1]

    @pl.when(g == 0)
    def _():
        bd = cd[0, 0]
        bi = ci[0, 0]
        for w in range(1, NW):
            dw = cd[w, 0]
            iw = ci[w, 0]
            better = dw < bd
            bd = jnp.where(better, dw, bd)
            bi = jnp.where(better, iw, bi)
        bmu_i = lax.shift_right_logical(bi, 7)
        bmu_j = lax.bitwise_and(bi, jnp.int32(N - 1))
        bsm[0] = bmu_i
        bsm[1] = bmu_j
        colj = lax.shift_right_logical(
            lax.broadcasted_iota(jnp.int32, (1, WCOL), 1), 8)
        dj = colj - bmu_j
        rj_ref[...] = jnp.exp(-(dj * dj).astype(jnp.float32) * inv_sig2)
        col = lax.broadcasted_iota(jnp.int32, (8, 128), 1)
        row0 = lax.broadcasted_iota(jnp.int32, (8, 128), 0)
        bmu_out[...] = jnp.where((row0 == 0) & (col == 0), bmu_i,
                                 jnp.where((row0 == 0) & (col == 1), bmu_j, 0))

    bmu_i = bsm[0]
    rows_i = g * BI + lax.broadcasted_iota(jnp.int32, (BI, 1), 0)
    di = rows_i - bmu_i
    ri = alpha_op * jnp.exp(-(di * di).astype(jnp.float32) * inv_sig2)
    rate = ri * rj_ref[...]
    wv = wr[...]
    out[...] = wv + rate * (xr[...] - wv)


_tc_update = pl.pallas_call(
    _tc_update_body,
    grid=(M // BI,),
    in_specs=[
        pl.BlockSpec((NW, NLANE), lambda g: (0, 0)),
        pl.BlockSpec((NW, NLANE), lambda g: (0, 0)),
        pl.BlockSpec((1, WCOL), lambda g: (0, 0)),
        pl.BlockSpec((1, 128), lambda g: (0, 0)),
        pl.BlockSpec((BI, WCOL), lambda g: (g, 0)),
    ],
    out_specs=[
        pl.BlockSpec((BI, WCOL), lambda g: (g, 0)),
        pl.BlockSpec((8, 128), lambda g: (0, 0)),
    ],
    out_shape=[
        jax.ShapeDtypeStruct((M, WCOL), jnp.float32),
        jax.ShapeDtypeStruct((8, 128), jnp.int32),
    ],
    scratch_shapes=[pltpu.SMEM((2,), jnp.int32),
                    pltpu.VMEM((1, WCOL), jnp.float32)],
)


def kernel(input_vector, weights, epoch):
    epoch_f = jnp.asarray(epoch, jnp.float32)
    lr = 1.0 - epoch_f / NUM_EPOCHS
    alpha_op = ALPHA * lr
    sigma_op = SIGMA * lr
    inv_sig2 = 1.0 / (sigma_op * sigma_op)
    params = jnp.zeros((1, 128), jnp.float32)
    params = params.at[0, 0].set(alpha_op).at[0, 1].set(inv_sig2)
    dists, idxs = _sc_search(weights, input_vector)
    w2 = weights.reshape(M, WCOL)
    xrep = jnp.tile(input_vector, N).reshape(1, WCOL)
    new_w2, bmu_pad = _tc_update(dists, idxs, xrep, params, w2)
    return bmu_pad[0, :2], new_w2.reshape(ROWS, DIM)


# speedup vs baseline: 1.5326x; 1.5326x over previous
import functools

import jax
import jax.numpy as jnp
from jax import lax
from jax.experimental import pallas as pl
from jax.experimental.pallas import tpu as pltpu
from jax.experimental.pallas import tpu_sc as plsc

M, N, DIM = 128, 128, 256
ALPHA = 0.3
SIGMA = max(M, N) / 2.0
NUM_EPOCHS = 100

ROWS = M * N
NC, NS, NLANE = 2, 16, 16
NW = NC * NS
RPW = ROWS // NW
NCH = DIM // NLANE
CH1 = 128
BI = 16
WCOL = N * DIM


def _mesh():
    return plsc.VectorSubcoreMesh(core_axis_name="c", subcore_axis_name="s")


@functools.partial(
    pl.kernel,
    mesh=_mesh(),
    compiler_params=pltpu.CompilerParams(needs_layout_passes=False),
    out_type=[
        jax.ShapeDtypeStruct((NW, NLANE), jnp.float32),
        jax.ShapeDtypeStruct((NW, NLANE), jnp.int32),
    ],
    scratch_types=[
        pltpu.VMEM((DIM,), jnp.float32),
        pltpu.VMEM((CH1, DIM), jnp.float32),
        pltpu.VMEM((CH1, DIM), jnp.float32),
        pltpu.VMEM((NLANE,), jnp.float32),
        pltpu.VMEM((NLANE,), jnp.int32),
        pltpu.SemaphoreType.DMA,
        pltpu.SemaphoreType.DMA,
    ],
)
def _sc_search(w_hbm, x_hbm, dist_out, idx_out, xv, buf0, buf1, sd, si,
               sem0, sem1):
    wid = lax.axis_index("s") * NC + lax.axis_index("c")
    base = wid * RPW
    pltpu.sync_copy(x_hbm, xv)
    xs = [xv[pl.ds(c * NLANE, NLANE)] for c in range(NCH)]
    bufs = (buf0, buf1)
    sems = (sem0, sem1)
    nchunk = RPW // CH1
    cps = [None] * nchunk
    cps[0] = pltpu.async_copy(w_hbm.at[pl.ds(base, CH1)], buf0, sem0)
    best = (jnp.float32(jnp.inf), jnp.int32(0))
    for k in range(nchunk):
        if k + 1 < nchunk:
            cps[k + 1] = pltpu.async_copy(
                w_hbm.at[pl.ds(base + (k + 1) * CH1, CH1)],
                bufs[(k + 1) % 2], sems[(k + 1) % 2])
        cps[k].wait()
        buf = bufs[k % 2]

        def row_body(r, carry, _k=k, _buf=buf):
            bd, bi = carry
            accs = [jnp.zeros((NLANE,), jnp.float32) for _ in range(4)]
            for c in range(NCH):
                d = _buf[r, pl.ds(c * NLANE, NLANE)] - xs[c]
                accs[c % 4] = accs[c % 4] + d * d
            acc = (accs[0] + accs[1]) + (accs[2] + accs[3])
            s = plsc.cumsum(acc)[NLANE - 1]
            better = s < bd
            gi = base + _k * CH1 + r
            return (jnp.where(better, s, bd), jnp.where(better, gi, bi))

        best = lax.fori_loop(0, CH1, row_body, best, unroll=4)
    sd[...] = jnp.full((NLANE,), best[0], jnp.float32)
    si[...] = jnp.full((NLANE,), best[1], jnp.int32)
    pltpu.sync_copy(sd, dist_out.at[wid])
    pltpu.sync_copy(si, idx_out.at[wid])


def _tc_update_body(cd, ci, xr, pr, wr, out, bmu_out, bsm, rj_ref):
    g = pl.program_id(0)
    alpha_op = pr[0, 0]
    inv_sig2 = pr[0, 1]

    @pl.when(g == 0)
    def _():
        bd = cd[0, 0]
        bi = ci[0, 0]
        for w in range(1, NW):
            dw = cd[w, 0]
            iw = ci[w, 0]
            better = dw < bd
            bd = jnp.where(better, dw, bd)
            bi = jnp.where(better, iw, bi)
        bmu_i = lax.shift_right_logical(bi, 7)
        bmu_j = lax.bitwise_and(bi, jnp.int32(N - 1))
        bsm[0] = bmu_i
        bsm[1] = bmu_j
        colj = lax.shift_right_logical(
            lax.broadcasted_iota(jnp.int32, (1, WCOL), 1), 8)
        dj = colj - bmu_j
        rj_ref[...] = jnp.exp(-(dj * dj).astype(jnp.float32) * inv_sig2)
        col = lax.broadcasted_iota(jnp.int32, (8, 128), 1)
        row0 = lax.broadcasted_iota(jnp.int32, (8, 128), 0)
        bmu_out[...] = jnp.where((row0 == 0) & (col == 0), bmu_i,
                                 jnp.where((row0 == 0) & (col == 1), bmu_j, 0))

    bmu_i = bsm[0]
    rows_i = g * BI + lax.broadcasted_iota(jnp.int32, (BI, 1), 0)
    di = rows_i - bmu_i
    ri = alpha_op * jnp.exp(-(di * di).astype(jnp.float32) * inv_sig2)
    rate = jnp.reshape(ri * rj_ref[...], (BI * N, DIM))
    wv = wr[...]
    out[...] = wv + rate * (xr[...] - wv)


_tc_update = pl.pallas_call(
    _tc_update_body,
    grid=(M // BI,),
    in_specs=[
        pl.BlockSpec((NW, NLANE), lambda g: (0, 0)),
        pl.BlockSpec((NW, NLANE), lambda g: (0, 0)),
        pl.BlockSpec((1, DIM), lambda g: (0, 0)),
        pl.BlockSpec((1, 128), lambda g: (0, 0)),
        pl.BlockSpec((BI * N, DIM), lambda g: (g, 0)),
    ],
    out_specs=[
        pl.BlockSpec((BI * N, DIM), lambda g: (g, 0)),
        pl.BlockSpec((8, 128), lambda g: (0, 0)),
    ],
    out_shape=[
        jax.ShapeDtypeStruct((ROWS, DIM), jnp.float32),
        jax.ShapeDtypeStruct((8, 128), jnp.int32),
    ],
    scratch_shapes=[pltpu.SMEM((2,), jnp.int32),
                    pltpu.VMEM((1, WCOL), jnp.float32)],
)


def kernel(input_vector, weights, epoch):
    epoch_f = jnp.asarray(epoch, jnp.float32)
    lr = 1.0 - epoch_f / NUM_EPOCHS
    alpha_op = ALPHA * lr
    sigma_op = SIGMA * lr
    inv_sig2 = 1.0 / (sigma_op * sigma_op)
    params = jnp.zeros((1, 128), jnp.float32)
    params = params.at[0, 0].set(alpha_op).at[0, 1].set(inv_sig2)
    dists, idxs = _sc_search(weights, input_vector)
    new_w, bmu_pad = _tc_update(dists, idxs, input_vector.reshape(1, DIM),
                                params, weights)
    return bmu_pad[0, :2], new_w
